# baseline (speedup 1.0000x reference)
import functools

import jax
import jax.numpy as jnp
from jax import lax
from jax.experimental import pallas as pl
from jax.experimental.pallas import tpu as pltpu
from jax.experimental.pallas import tpu_sc as plsc

N, D, H, E = 32768, 768, 128, 64
BN = 4096
BNC = 8192
NC, NS, L = 2, 16, 16
NW = NC * NS
TPW = N // NW


def _logits_t_body(x_ref, w1_ref, b1_ref, w2_ref, b2_ref, out_ref):
    h = jnp.dot(x_ref[...], w1_ref[...], preferred_element_type=jnp.float32)
    h = jnp.maximum(h + b1_ref[...], 0.0)
    logits_t = lax.dot_general(w2_ref[...], h, (((0,), (1,)), ((), ())),
                               preferred_element_type=jnp.float32)
    out_ref[...] = logits_t + b2_ref[...]


def _logits_t(x, W1, b1, W2, b2):
    return pl.pallas_call(
        _logits_t_body,
        grid=(N // BN,),
        in_specs=[
            pl.BlockSpec((BN, D), lambda i: (i, 0)),
            pl.BlockSpec((D, H), lambda i: (0, 0)),
            pl.BlockSpec((1, H), lambda i: (0, 0)),
            pl.BlockSpec((H, E), lambda i: (0, 0)),
            pl.BlockSpec((E, 1), lambda i: (0, 0)),
        ],
        out_specs=pl.BlockSpec((E, BN), lambda i: (0, i)),
        out_shape=jax.ShapeDtypeStruct((E, N), jnp.float32),
    )(x, W1, b1.reshape(1, H), W2, b2.reshape(E, 1))


_SC_MESH = plsc.VectorSubcoreMesh(core_axis_name="c", subcore_axis_name="s")


@functools.partial(
    pl.kernel,
    mesh=_SC_MESH,
    out_type=jax.ShapeDtypeStruct((8, N), jnp.float32),
    scratch_types=[
        pltpu.VMEM((E, TPW), jnp.float32),
        pltpu.VMEM((5, TPW), jnp.float32),
    ],
)
def _sc_gate(logT, out8, buf, obuf):
    wid = lax.axis_index("s") * NC + lax.axis_index("c")
    base = wid * TPW
    pltpu.sync_copy(logT.at[:, pl.ds(base, TPW)], buf)

    def group(g, _):
        t0 = g * L
        m1 = buf[0, pl.ds(t0, L)]
        i1 = jnp.zeros((L,), jnp.float32)
        m2 = jnp.full((L,), -jnp.inf, jnp.float32)
        i2 = jnp.full((L,), float(E), jnp.float32)
        for e in range(1, E):
            v = buf[e, pl.ds(t0, L)]
            ef = jnp.full((L,), float(e), jnp.float32)
            gt1 = v > m1
            gt2 = v > m2
            m2, i2 = (jnp.where(gt1, m1, jnp.where(gt2, v, m2)),
                      jnp.where(gt1, i1, jnp.where(gt2, ef, i2)))
            m1, i1 = jnp.where(gt1, v, m1), jnp.where(gt1, ef, i1)
        m = jnp.maximum(m1, 0.0)
        e1 = jnp.exp(m1 - m)
        e2 = jnp.exp(m2 - m)
        zv = jnp.exp(0.0 - m)
        rden = 1.0 / (e1 + e2 + (E - 2) * zv)
        vals = (e1 * rden, e2 * rden, zv * rden, i1, i2)
        for k, val in enumerate(vals):
            obuf[k, pl.ds(t0, L)] = val
        return 0

    lax.fori_loop(0, TPW // L, group, 0)
    for k in range(5):
        pltpu.sync_copy(obuf.at[pl.ds(k, 1)],
                        out8.at[pl.ds(k, 1), pl.ds(base, TPW)])


def _expand_body(c_ref, out_ref):
    c = c_ref[...]

    def bcast(k):
        sk = (lax.broadcasted_iota(jnp.int32, (8, E), 0) == k)
        return lax.dot_general(c, sk.astype(jnp.float32),
                               (((0,), (0,)), ((), ())),
                               preferred_element_type=jnp.float32)

    bw1, bw2, bzv, bi1, bi2 = (bcast(k) for k in range(5))
    colf = lax.broadcasted_iota(jnp.int32, (BNC, E), 1).astype(jnp.float32)
    out_ref[...] = jnp.where(colf == bi1, bw1,
                             jnp.where(colf == bi2, bw2, bzv))


def _expand(c):
    return pl.pallas_call(
        _expand_body,
        grid=(N // BNC,),
        in_specs=[pl.BlockSpec((8, BNC), lambda i: (0, i))],
        out_specs=pl.BlockSpec((BNC, E), lambda i: (i, 0)),
        out_shape=jax.ShapeDtypeStruct((N, E), jnp.float32),
    )(c)


@jax.jit
def kernel(x, W1, b1, W2, b2):
    logT = _logits_t(x, W1, b1, W2, b2)
    return _expand(_sc_gate(logT))


# speedup vs baseline: 1.0025x; 1.0025x over previous
import functools

import jax
import jax.numpy as jnp
from jax import lax
from jax.experimental import pallas as pl
from jax.experimental.pallas import tpu as pltpu
from jax.experimental.pallas import tpu_sc as plsc

N, D, H, E = 32768, 768, 128, 64
BN = 4096
BNC = 8192
NC, NS, L = 2, 16, 16
NW = NC * NS
TPW = N // NW


def _logits_t_body(x_ref, w1_ref, b1_ref, w2_ref, b2_ref, out_ref):
    h = jnp.dot(x_ref[...], w1_ref[...], preferred_element_type=jnp.float32)
    h = jnp.maximum(h + b1_ref[...], 0.0)
    logits_t = lax.dot_general(w2_ref[...], h, (((0,), (1,)), ((), ())),
                               preferred_element_type=jnp.float32)
    out_ref[...] = logits_t + b2_ref[...]


def _logits_t(x, W1, b1, W2, b2):
    return pl.pallas_call(
        _logits_t_body,
        grid=(N // BN,),
        in_specs=[
            pl.BlockSpec((BN, D), lambda i: (i, 0)),
            pl.BlockSpec((D, H), lambda i: (0, 0)),
            pl.BlockSpec((1, H), lambda i: (0, 0)),
            pl.BlockSpec((H, E), lambda i: (0, 0)),
            pl.BlockSpec((E, 1), lambda i: (0, 0)),
        ],
        out_specs=pl.BlockSpec((E, BN), lambda i: (0, i)),
        out_shape=jax.ShapeDtypeStruct((E, N), jnp.float32),
    )(x, W1, b1.reshape(1, H), W2, b2.reshape(E, 1))


_SC_MESH = plsc.VectorSubcoreMesh(core_axis_name="c", subcore_axis_name="s")


@functools.partial(
    pl.kernel,
    mesh=_SC_MESH,
    out_type=jax.ShapeDtypeStruct((8, N), jnp.float32),
    scratch_types=[
        pltpu.VMEM((E, TPW), jnp.float32),
        pltpu.VMEM((5, TPW), jnp.float32),
    ],
)
def _sc_gate(logT, out8, buf, obuf):
    wid = lax.axis_index("s") * NC + lax.axis_index("c")
    base = wid * TPW
    pltpu.sync_copy(logT.at[:, pl.ds(base, TPW)], buf)

    def group(g, _):
        t0 = g * L
        m1 = buf[0, pl.ds(t0, L)]
        i1 = jnp.zeros((L,), jnp.float32)
        m2 = jnp.full((L,), -jnp.inf, jnp.float32)
        i2 = jnp.full((L,), float(E), jnp.float32)
        for e in range(1, E):
            v = buf[e, pl.ds(t0, L)]
            ef = jnp.full((L,), float(e), jnp.float32)
            gt1 = v > m1
            gt2 = v > m2
            m2, i2 = (jnp.where(gt1, m1, jnp.where(gt2, v, m2)),
                      jnp.where(gt1, i1, jnp.where(gt2, ef, i2)))
            m1, i1 = jnp.where(gt1, v, m1), jnp.where(gt1, ef, i1)
        m = jnp.maximum(m1, 0.0)
        e1 = jnp.exp(m1 - m)
        e2 = jnp.exp(m2 - m)
        zv = jnp.exp(0.0 - m)
        rden = 1.0 / (e1 + e2 + (E - 2) * zv)
        vals = (e1 * rden, e2 * rden, zv * rden, i1, i2)
        for k, val in enumerate(vals):
            obuf[k, pl.ds(t0, L)] = val
        return 0

    lax.fori_loop(0, TPW // L, group, 0)
    pltpu.sync_copy(obuf, out8.at[pl.ds(0, 5), pl.ds(base, TPW)])


def _expand_body(c_ref, out_ref):
    c = c_ref[...]

    def bcast(k):
        sk = (lax.broadcasted_iota(jnp.int32, (8, E), 0) == k)
        return lax.dot_general(c, sk.astype(jnp.float32),
                               (((0,), (0,)), ((), ())),
                               preferred_element_type=jnp.float32)

    bw1, bw2, bzv, bi1, bi2 = (bcast(k) for k in range(5))
    colf = lax.broadcasted_iota(jnp.int32, (BNC, E), 1).astype(jnp.float32)
    out_ref[...] = jnp.where(colf == bi1, bw1,
                             jnp.where(colf == bi2, bw2, bzv))


def _expand(c):
    return pl.pallas_call(
        _expand_body,
        grid=(N // BNC,),
        in_specs=[pl.BlockSpec((8, BNC), lambda i: (0, i))],
        out_specs=pl.BlockSpec((BNC, E), lambda i: (i, 0)),
        out_shape=jax.ShapeDtypeStruct((N, E), jnp.float32),
    )(c)


@jax.jit
def kernel(x, W1, b1, W2, b2):
    logT = _logits_t(x, W1, b1, W2, b2)
    return _expand(_sc_gate(logT))
